# baseline (speedup 1.0000x reference)
import functools

import jax
import jax.numpy as jnp
from jax import lax
from jax.experimental import pallas as pl
from jax.experimental.pallas import tpu as pltpu
from jax.experimental.pallas import tpu_sc as plsc

N_NODES = 10000
N_EDGES = 160000
D_IN = 256
D_OUT = 256
HALF = 128

NC = 2
NS = 16
CHUNK = 128
IDXBLK = 8
NBLK = 10
NCHUNK = NBLK * IDXBLK
EPT_PAD = NCHUNK * CHUNK
NPAD = 10240
ROWS_PER_TILE = NPAD // NS
ZROWS = 128
TRASH = N_NODES


def _sc_segment_sum(feat_flat, src4, dst4):
    mesh = plsc.VectorSubcoreMesh(
        core_axis_name="c", subcore_axis_name="s", num_cores=NC, num_subcores=NS
    )

    @functools.partial(
        pl.kernel,
        out_type=(
            jax.ShapeDtypeStruct((NC * NPAD, HALF), jnp.float32),
            jax.ShapeDtypeStruct((NC * NPAD, HALF), jnp.float32),
        ),
        mesh=mesh,
        scratch_types=[
            pltpu.VMEM((IDXBLK, CHUNK), jnp.int32),
            pltpu.VMEM((IDXBLK, CHUNK), jnp.int32),
            pltpu.VMEM((CHUNK, HALF), jnp.float32),
            pltpu.VMEM((CHUNK, HALF), jnp.float32),
            pltpu.VMEM_SHARED((NPAD, HALF), jnp.float32),
            pltpu.SemaphoreType.DMA,
            pltpu.SemaphoreType.DMA,
            pltpu.SemaphoreType.DMA,
            pltpu.SemaphoreType.DMA,
        ],
    )
    def k(feat_hbm, src_hbm, dst_hbm, ns_out, cnt_out, src_v, dst_v,
          gbuf, gbuf2, acc, gsem0, gsem1, ssem0, ssem1):
        c = lax.axis_index("c")
        s = lax.axis_index("s")
        tile_rows = s * ROWS_PER_TILE
        out_base = c * NPAD + tile_rows

        zero16 = jnp.zeros((16,), jnp.float32)
        one16 = jnp.ones((16,), jnp.float32)

        def fill(val16):
            def go(i, _):
                for g in range(HALF // 16):
                    gbuf[i, pl.ds(g * 16, 16)] = val16
                return 0
            lax.fori_loop(0, ZROWS, go, 0)

        def zero_acc():
            for b in range(ROWS_PER_TILE // ZROWS):
                rows = pl.ds(tile_rows + b * ZROWS, ZROWS)
                pltpu.sync_copy(gbuf, acc.at[rows])

        def write_out(out_ref):
            for b in range(ROWS_PER_TILE // ZROWS):
                rows = pl.ds(tile_rows + b * ZROWS, ZROWS)
                orows = pl.ds(out_base + b * ZROWS, ZROWS)
                pltpu.sync_copy(acc.at[rows], out_ref.at[orows])

        with jax.named_scope("p1_zero"):
            fill(zero16)
            zero_acc()
            plsc.subcore_barrier()

        src_off = (c * N_NODES).astype(jnp.int32)

        gbufs = (gbuf, gbuf2)
        gsems = (gsem0, gsem1)
        ssems = (ssem0, ssem1)

        def block1(blk, _):
            pltpu.sync_copy(src_hbm.at[s, blk], src_v)
            pltpu.sync_copy(dst_hbm.at[s, blk], dst_v)

            def adj(i, _):
                for g in range(CHUNK // 16):
                    sl = pl.ds(g * 16, 16)
                    src_v[i, sl] = src_v[i, sl] + src_off
                return 0

            lax.fori_loop(0, IDXBLK, adj, 0)

            gdesc = [None] * IDXBLK
            sdesc = [None] * IDXBLK
            gdesc[0] = pltpu.async_copy(
                feat_hbm.at[src_v.at[0]], gbufs[0], gsems[0])
            for j in range(IDXBLK):
                b = j % 2
                gdesc[j].wait()
                if j >= 1:
                    sdesc[j - 1].wait()
                if j < IDXBLK - 1:
                    gdesc[j + 1] = pltpu.async_copy(
                        feat_hbm.at[src_v.at[j + 1]], gbufs[1 - b],
                        gsems[1 - b])
                sdesc[j] = pltpu.async_copy(
                    gbufs[b], acc.at[dst_v.at[j]], ssems[b], add=True)
            sdesc[IDXBLK - 1].wait()
            return 0

        with jax.named_scope("p1_main"):
            lax.fori_loop(0, NBLK, block1, 0)
            plsc.subcore_barrier()
        with jax.named_scope("p1_out"):
            write_out(ns_out)

        with jax.named_scope("p2_zero"):
            fill(zero16)
            zero_acc()
            fill(one16)
            plsc.subcore_barrier()

        def block2(blk, _):
            blk2 = blk * NC + c
            pltpu.sync_copy(dst_hbm.at[s, blk2], dst_v)
            descs = [
                pltpu.async_copy(gbuf, acc.at[dst_v.at[j]], ssem0, add=True)
                for j in range(IDXBLK)
            ]
            for d in descs:
                d.wait()
            return 0

        with jax.named_scope("p2_main"):
            lax.fori_loop(0, NBLK // NC, block2, 0)
            plsc.subcore_barrier()
        with jax.named_scope("p2_out"):
            write_out(cnt_out)

    return k(feat_flat, src4, dst4)


def _tc_finish(features, ns, cnt, W_enc, b_enc2, W_tr):
    BN = 400
    grid = (N_NODES // BN,)

    def body(x_ref, ns_ref, cnt_ref, wenc_ref, benc_ref, wtr_ref, out_ref,
             bw_scr, bb_scr):
        i = pl.program_id(0)

        @pl.when(i == 0)
        def _():
            wt_bot = wtr_ref[D_IN:, :]
            bw_scr[...] = jnp.dot(
                wenc_ref[...], wt_bot, preferred_element_type=jnp.float32
            )
            bb_scr[...] = jnp.dot(
                benc_ref[...], wt_bot, preferred_element_type=jnp.float32
            )

        x = x_ref[...]
        nsum = jnp.concatenate([ns_ref[0], ns_ref[1]], axis=1)
        cnt = cnt_ref[0, :, 0:1] + cnt_ref[1, :, 0:1]
        pooled = nsum / jnp.maximum(cnt, 1.0)
        acc = jnp.dot(x, wtr_ref[:D_IN, :], preferred_element_type=jnp.float32)
        acc = acc + jnp.dot(pooled, bw_scr[...],
                            preferred_element_type=jnp.float32)
        out_ref[...] = jnp.maximum(acc + bb_scr[...], 0.0)

    return pl.pallas_call(
        body,
        grid=grid,
        in_specs=[
            pl.BlockSpec((BN, D_IN), lambda i: (i, 0)),
            pl.BlockSpec((NC, BN, HALF), lambda i: (0, i, 0)),
            pl.BlockSpec((NC, BN, HALF), lambda i: (0, i, 0)),
            pl.BlockSpec((D_IN, D_OUT), lambda i: (0, 0)),
            pl.BlockSpec((1, D_OUT), lambda i: (0, 0)),
            pl.BlockSpec((D_IN + D_OUT, D_OUT), lambda i: (0, 0)),
        ],
        out_specs=pl.BlockSpec((BN, D_OUT), lambda i: (i, 0)),
        out_shape=jax.ShapeDtypeStruct((N_NODES, D_OUT), jnp.float32),
        scratch_shapes=[
            pltpu.VMEM((D_OUT, D_OUT), jnp.float32),
            pltpu.VMEM((1, D_OUT), jnp.float32),
        ],
    )(features, ns, cnt, W_enc, b_enc2, W_tr)


@jax.jit
def kernel(features, edge_index, W_enc, b_enc, W_tr):
    dst = edge_index[0].astype(jnp.int32)
    src = edge_index[1].astype(jnp.int32)

    pad = NS * EPT_PAD - N_EDGES
    src4 = jnp.concatenate(
        [src, jnp.zeros((pad,), jnp.int32)]).reshape(NS, NBLK, IDXBLK, CHUNK)
    dst4 = jnp.concatenate(
        [dst, jnp.full((pad,), TRASH, jnp.int32)]).reshape(NS, NBLK, IDXBLK, CHUNK)

    feat_flat = features.reshape(N_NODES, NC, HALF).transpose(1, 0, 2).reshape(
        NC * N_NODES, HALF)

    ns, cnt = _sc_segment_sum(feat_flat, src4, dst4)

    return _tc_finish(features, ns.reshape(NC, NPAD, HALF),
                      cnt.reshape(NC, NPAD, HALF),
                      W_enc, b_enc.reshape(1, D_OUT), W_tr)


# speedup vs baseline: 1.0378x; 1.0378x over previous
import functools

import jax
import jax.numpy as jnp
from jax import lax
from jax.experimental import pallas as pl
from jax.experimental.pallas import tpu as pltpu
from jax.experimental.pallas import tpu_sc as plsc

N_NODES = 10000
N_EDGES = 160000
D_IN = 256
D_OUT = 256
HALF = 128

NC = 2
NS = 16
CHUNK = 128
IDXBLK = 8
NBLK = 10
NCHUNK = NBLK * IDXBLK
EPT_PAD = NCHUNK * CHUNK
NPAD = 10240
ROWS_PER_TILE = NPAD // NS
ZROWS = 128
TRASH = N_NODES


def _sc_segment_sum(feat_flat, src4, dst4):
    mesh = plsc.VectorSubcoreMesh(
        core_axis_name="c", subcore_axis_name="s", num_cores=NC, num_subcores=NS
    )

    @functools.partial(
        pl.kernel,
        out_type=(
            jax.ShapeDtypeStruct((NC * NPAD, HALF), jnp.float32),
            jax.ShapeDtypeStruct((NC * NPAD, HALF), jnp.float32),
        ),
        mesh=mesh,
        scratch_types=[
            pltpu.VMEM((IDXBLK, CHUNK), jnp.int32),
            pltpu.VMEM((IDXBLK, CHUNK), jnp.int32),
            pltpu.VMEM((CHUNK, HALF), jnp.float32),
            pltpu.VMEM((CHUNK, HALF), jnp.float32),
            pltpu.VMEM_SHARED((NPAD, HALF), jnp.float32),
            pltpu.SemaphoreType.DMA,
            pltpu.SemaphoreType.DMA,
            pltpu.SemaphoreType.DMA,
            pltpu.SemaphoreType.DMA,
        ],
    )
    def k(feat_hbm, src_hbm, dst_hbm, ns_out, cnt_out, src_v, dst_v,
          gbuf, gbuf2, acc, gsem0, gsem1, ssem0, ssem1):
        c = lax.axis_index("c")
        s = lax.axis_index("s")
        tile_rows = s * ROWS_PER_TILE
        out_base = c * NPAD + tile_rows

        zero16 = jnp.zeros((16,), jnp.float32)
        one16 = jnp.ones((16,), jnp.float32)

        def fill(val16):
            def go(i, _):
                for g in range(HALF // 16):
                    gbuf[i, pl.ds(g * 16, 16)] = val16
                return 0
            lax.fori_loop(0, ZROWS, go, 0)

        def zero_acc():
            for b in range(ROWS_PER_TILE // ZROWS):
                rows = pl.ds(tile_rows + b * ZROWS, ZROWS)
                pltpu.sync_copy(gbuf, acc.at[rows])

        def write_out(out_ref):
            for b in range(ROWS_PER_TILE // ZROWS):
                rows = pl.ds(tile_rows + b * ZROWS, ZROWS)
                orows = pl.ds(out_base + b * ZROWS, ZROWS)
                pltpu.sync_copy(acc.at[rows], out_ref.at[orows])

        with jax.named_scope("p1_zero"):
            fill(zero16)
            zero_acc()
            plsc.subcore_barrier()

        src_off = (c * N_NODES).astype(jnp.int32)

        gbufs = (gbuf, gbuf2)
        gsems = (gsem0, gsem1)
        ssems = (ssem0, ssem1)

        def block1(blk, _):
            pltpu.sync_copy(src_hbm.at[s, blk], src_v)
            pltpu.sync_copy(dst_hbm.at[s, blk], dst_v)

            def adj(i, _):
                for g in range(CHUNK // 16):
                    sl = pl.ds(g * 16, 16)
                    src_v[i, sl] = src_v[i, sl] + src_off
                return 0

            lax.fori_loop(0, IDXBLK, adj, 0)

            gdesc = [None] * IDXBLK
            sdesc = [None] * IDXBLK
            gdesc[0] = pltpu.async_copy(
                feat_hbm.at[src_v.at[0]], gbufs[0], gsems[0])
            for j in range(IDXBLK):
                b = j % 2
                if j >= 1:
                    sdesc[j - 1].wait()
                if j < IDXBLK - 1:
                    gdesc[j + 1] = pltpu.async_copy(
                        feat_hbm.at[src_v.at[j + 1]], gbufs[1 - b],
                        gsems[1 - b])
                gdesc[j].wait()
                sdesc[j] = pltpu.async_copy(
                    gbufs[b], acc.at[dst_v.at[j]], ssems[b], add=True)
            sdesc[IDXBLK - 1].wait()
            return 0

        with jax.named_scope("p1_main"):
            lax.fori_loop(0, NBLK, block1, 0)
            plsc.subcore_barrier()
        with jax.named_scope("p1_out"):
            write_out(ns_out)

        with jax.named_scope("p2_zero"):
            fill(zero16)
            zero_acc()
            fill(one16)
            plsc.subcore_barrier()

        def block2(blk, _):
            blk2 = blk * NC + c
            pltpu.sync_copy(dst_hbm.at[s, blk2], dst_v)
            descs = [
                pltpu.async_copy(gbuf, acc.at[dst_v.at[j]], ssem0, add=True)
                for j in range(IDXBLK)
            ]
            for d in descs:
                d.wait()
            return 0

        with jax.named_scope("p2_main"):
            lax.fori_loop(0, NBLK // NC, block2, 0)
            plsc.subcore_barrier()
        with jax.named_scope("p2_out"):
            write_out(cnt_out)

    return k(feat_flat, src4, dst4)


def _tc_finish(features, ns, cnt, W_enc, b_enc2, W_tr):
    BN = 400
    grid = (N_NODES // BN,)

    def body(x_ref, ns_ref, cnt_ref, wenc_ref, benc_ref, wtr_ref, out_ref,
             bw_scr, bb_scr):
        i = pl.program_id(0)

        @pl.when(i == 0)
        def _():
            wt_bot = wtr_ref[D_IN:, :]
            bw_scr[...] = jnp.dot(
                wenc_ref[...], wt_bot, preferred_element_type=jnp.float32
            )
            bb_scr[...] = jnp.dot(
                benc_ref[...], wt_bot, preferred_element_type=jnp.float32
            )

        x = x_ref[...]
        nsum = jnp.concatenate([ns_ref[0], ns_ref[1]], axis=1)
        cnt = cnt_ref[0, :, 0:1] + cnt_ref[1, :, 0:1]
        pooled = nsum / jnp.maximum(cnt, 1.0)
        acc = jnp.dot(x, wtr_ref[:D_IN, :], preferred_element_type=jnp.float32)
        acc = acc + jnp.dot(pooled, bw_scr[...],
                            preferred_element_type=jnp.float32)
        out_ref[...] = jnp.maximum(acc + bb_scr[...], 0.0)

    return pl.pallas_call(
        body,
        grid=grid,
        in_specs=[
            pl.BlockSpec((BN, D_IN), lambda i: (i, 0)),
            pl.BlockSpec((NC, BN, HALF), lambda i: (0, i, 0)),
            pl.BlockSpec((NC, BN, HALF), lambda i: (0, i, 0)),
            pl.BlockSpec((D_IN, D_OUT), lambda i: (0, 0)),
            pl.BlockSpec((1, D_OUT), lambda i: (0, 0)),
            pl.BlockSpec((D_IN + D_OUT, D_OUT), lambda i: (0, 0)),
        ],
        out_specs=pl.BlockSpec((BN, D_OUT), lambda i: (i, 0)),
        out_shape=jax.ShapeDtypeStruct((N_NODES, D_OUT), jnp.float32),
        scratch_shapes=[
            pltpu.VMEM((D_OUT, D_OUT), jnp.float32),
            pltpu.VMEM((1, D_OUT), jnp.float32),
        ],
    )(features, ns, cnt, W_enc, b_enc2, W_tr)


@jax.jit
def kernel(features, edge_index, W_enc, b_enc, W_tr):
    dst = edge_index[0].astype(jnp.int32)
    src = edge_index[1].astype(jnp.int32)

    pad = NS * EPT_PAD - N_EDGES
    src4 = jnp.concatenate(
        [src, jnp.zeros((pad,), jnp.int32)]).reshape(NS, NBLK, IDXBLK, CHUNK)
    dst4 = jnp.concatenate(
        [dst, jnp.full((pad,), TRASH, jnp.int32)]).reshape(NS, NBLK, IDXBLK, CHUNK)

    feat_flat = features.reshape(N_NODES, NC, HALF).transpose(1, 0, 2).reshape(
        NC * N_NODES, HALF)

    ns, cnt = _sc_segment_sum(feat_flat, src4, dst4)

    return _tc_finish(features, ns.reshape(NC, NPAD, HALF),
                      cnt.reshape(NC, NPAD, HALF),
                      W_enc, b_enc.reshape(1, D_OUT), W_tr)


# speedup vs baseline: 1.0550x; 1.0167x over previous
import functools

import jax
import jax.numpy as jnp
from jax import lax
from jax.experimental import pallas as pl
from jax.experimental.pallas import tpu as pltpu
from jax.experimental.pallas import tpu_sc as plsc

N_NODES = 10000
N_EDGES = 160000
D_IN = 256
D_OUT = 256
HALF = 128

NC = 2
NS = 16
CHUNK = 128
IDXBLK = 16
NBLK = 5
NCHUNK = NBLK * IDXBLK
EPT_PAD = NCHUNK * CHUNK
NPAD = 10240
ROWS_PER_TILE = NPAD // NS
ZROWS = 128
TRASH = N_NODES


def _sc_segment_sum(feat_flat, src4, dst4):
    mesh = plsc.VectorSubcoreMesh(
        core_axis_name="c", subcore_axis_name="s", num_cores=NC, num_subcores=NS
    )

    @functools.partial(
        pl.kernel,
        out_type=(
            jax.ShapeDtypeStruct((NC * NPAD, HALF), jnp.float32),
            jax.ShapeDtypeStruct((NC * NPAD, HALF), jnp.float32),
        ),
        mesh=mesh,
        scratch_types=[
            pltpu.VMEM((IDXBLK, CHUNK), jnp.int32),
            pltpu.VMEM((IDXBLK, CHUNK), jnp.int32),
            pltpu.VMEM((CHUNK, HALF), jnp.float32),
            pltpu.VMEM((CHUNK, HALF), jnp.float32),
            pltpu.VMEM_SHARED((NPAD, HALF), jnp.float32),
            pltpu.SemaphoreType.DMA,
            pltpu.SemaphoreType.DMA,
            pltpu.SemaphoreType.DMA,
            pltpu.SemaphoreType.DMA,
        ],
    )
    def k(feat_hbm, src_hbm, dst_hbm, ns_out, cnt_out, src_v, dst_v,
          gbuf, gbuf2, acc, gsem0, gsem1, ssem0, ssem1):
        c = lax.axis_index("c")
        s = lax.axis_index("s")
        tile_rows = s * ROWS_PER_TILE
        out_base = c * NPAD + tile_rows

        zero16 = jnp.zeros((16,), jnp.float32)
        one16 = jnp.ones((16,), jnp.float32)

        def fill(val16):
            def go(i, _):
                for g in range(HALF // 16):
                    gbuf[i, pl.ds(g * 16, 16)] = val16
                return 0
            lax.fori_loop(0, ZROWS, go, 0)

        def zero_acc():
            for b in range(ROWS_PER_TILE // ZROWS):
                rows = pl.ds(tile_rows + b * ZROWS, ZROWS)
                pltpu.sync_copy(gbuf, acc.at[rows])

        def write_out(out_ref):
            for b in range(ROWS_PER_TILE // ZROWS):
                rows = pl.ds(tile_rows + b * ZROWS, ZROWS)
                orows = pl.ds(out_base + b * ZROWS, ZROWS)
                pltpu.sync_copy(acc.at[rows], out_ref.at[orows])

        with jax.named_scope("p1_zero"):
            fill(zero16)
            zero_acc()
            plsc.subcore_barrier()

        src_off = (c * N_NODES).astype(jnp.int32)

        gbufs = (gbuf, gbuf2)
        gsems = (gsem0, gsem1)
        ssems = (ssem0, ssem1)

        def block1(blk, _):
            pltpu.sync_copy(src_hbm.at[s, blk], src_v)
            pltpu.sync_copy(dst_hbm.at[s, blk], dst_v)

            def adj(i, _):
                for g in range(CHUNK // 16):
                    sl = pl.ds(g * 16, 16)
                    src_v[i, sl] = src_v[i, sl] + src_off
                return 0

            lax.fori_loop(0, IDXBLK, adj, 0)

            gdesc = [None] * IDXBLK
            sdesc = [None] * IDXBLK
            gdesc[0] = pltpu.async_copy(
                feat_hbm.at[src_v.at[0]], gbufs[0], gsems[0])
            for j in range(IDXBLK):
                b = j % 2
                if j >= 1:
                    sdesc[j - 1].wait()
                if j < IDXBLK - 1:
                    gdesc[j + 1] = pltpu.async_copy(
                        feat_hbm.at[src_v.at[j + 1]], gbufs[1 - b],
                        gsems[1 - b])
                gdesc[j].wait()
                sdesc[j] = pltpu.async_copy(
                    gbufs[b], acc.at[dst_v.at[j]], ssems[b], add=True)
            sdesc[IDXBLK - 1].wait()
            return 0

        with jax.named_scope("p1_main"):
            lax.fori_loop(0, NBLK, block1, 0)
            plsc.subcore_barrier()
        with jax.named_scope("p1_out"):
            write_out(ns_out)

        with jax.named_scope("p2_zero"):
            fill(zero16)
            zero_acc()
            fill(one16)
            plsc.subcore_barrier()

        def block2(blk, _):
            pltpu.sync_copy(dst_hbm.at[s, blk], dst_v)
            for par in range(NC):
                @pl.when(c == par)
                def _():
                    descs = [
                        pltpu.async_copy(gbuf, acc.at[dst_v.at[j]], ssem0,
                                         add=True)
                        for j in range(par, IDXBLK, NC)
                    ]
                    for d in descs:
                        d.wait()
            return 0

        with jax.named_scope("p2_main"):
            lax.fori_loop(0, NBLK, block2, 0)
            plsc.subcore_barrier()
        with jax.named_scope("p2_out"):
            write_out(cnt_out)

    return k(feat_flat, src4, dst4)


def _tc_finish(features, ns, cnt, W_enc, b_enc2, W_tr):
    BN = 400
    grid = (N_NODES // BN,)

    def body(x_ref, ns_ref, cnt_ref, wenc_ref, benc_ref, wtr_ref, out_ref,
             bw_scr, bb_scr):
        i = pl.program_id(0)

        @pl.when(i == 0)
        def _():
            wt_bot = wtr_ref[D_IN:, :]
            bw_scr[...] = jnp.dot(
                wenc_ref[...], wt_bot, preferred_element_type=jnp.float32
            )
            bb_scr[...] = jnp.dot(
                benc_ref[...], wt_bot, preferred_element_type=jnp.float32
            )

        x = x_ref[...]
        nsum = jnp.concatenate([ns_ref[0], ns_ref[1]], axis=1)
        cnt = cnt_ref[0, :, 0:1] + cnt_ref[1, :, 0:1]
        pooled = nsum / jnp.maximum(cnt, 1.0)
        acc = jnp.dot(x, wtr_ref[:D_IN, :], preferred_element_type=jnp.float32)
        acc = acc + jnp.dot(pooled, bw_scr[...],
                            preferred_element_type=jnp.float32)
        out_ref[...] = jnp.maximum(acc + bb_scr[...], 0.0)

    return pl.pallas_call(
        body,
        grid=grid,
        in_specs=[
            pl.BlockSpec((BN, D_IN), lambda i: (i, 0)),
            pl.BlockSpec((NC, BN, HALF), lambda i: (0, i, 0)),
            pl.BlockSpec((NC, BN, HALF), lambda i: (0, i, 0)),
            pl.BlockSpec((D_IN, D_OUT), lambda i: (0, 0)),
            pl.BlockSpec((1, D_OUT), lambda i: (0, 0)),
            pl.BlockSpec((D_IN + D_OUT, D_OUT), lambda i: (0, 0)),
        ],
        out_specs=pl.BlockSpec((BN, D_OUT), lambda i: (i, 0)),
        out_shape=jax.ShapeDtypeStruct((N_NODES, D_OUT), jnp.float32),
        scratch_shapes=[
            pltpu.VMEM((D_OUT, D_OUT), jnp.float32),
            pltpu.VMEM((1, D_OUT), jnp.float32),
        ],
    )(features, ns, cnt, W_enc, b_enc2, W_tr)


@jax.jit
def kernel(features, edge_index, W_enc, b_enc, W_tr):
    dst = edge_index[0].astype(jnp.int32)
    src = edge_index[1].astype(jnp.int32)

    pad = NS * EPT_PAD - N_EDGES
    src4 = jnp.concatenate(
        [src, jnp.zeros((pad,), jnp.int32)]).reshape(NS, NBLK, IDXBLK, CHUNK)
    dst4 = jnp.concatenate(
        [dst, jnp.full((pad,), TRASH, jnp.int32)]).reshape(NS, NBLK, IDXBLK, CHUNK)

    feat_flat = features.reshape(N_NODES, NC, HALF).transpose(1, 0, 2).reshape(
        NC * N_NODES, HALF)

    ns, cnt = _sc_segment_sum(feat_flat, src4, dst4)

    return _tc_finish(features, ns.reshape(NC, NPAD, HALF),
                      cnt.reshape(NC, NPAD, HALF),
                      W_enc, b_enc.reshape(1, D_OUT), W_tr)
